# Initial kernel scaffold; baseline (speedup 1.0000x reference)
import jax
import jax.numpy as jnp
from jax.experimental import pallas as pl


def kernel(x_img, y, w_cnn, b_cnn, emb_raw, w_emb, b_emb, struc):
    raise NotImplementedError("write your pallas kernel here")


# speedup vs baseline: 1.1956x; 1.1956x over previous
import functools

import jax
import jax.numpy as jnp
from jax.experimental import pallas as pl
from jax.experimental.pallas import tpu as pltpu

_VMEM_LIMIT = 48 * 1024 * 1024


def _fit_tile(dim, pref):
    t = max(1, min(pref, dim))
    while dim % t != 0:
        t //= 2
    return max(t, 1)


def _emb_kernel(raw_ref, w_ref, b_ref, emb_ref):
    raw = raw_ref[...].astype(jnp.bfloat16)
    w = w_ref[...].astype(jnp.bfloat16)
    proj = jnp.dot(raw, w, preferred_element_type=jnp.float32) + b_ref[...]
    ss = jnp.sum(proj * proj, axis=1, keepdims=True)
    emb_ref[...] = (proj * jax.lax.rsqrt(jnp.maximum(ss, 1e-24))).astype(emb_ref.dtype)


def _ce_kernel(x_ref, w_ref, b_ref, emb_ref, y_ref, ce_ref, *, inv_temperature, c, hw):
    x = x_ref[...]
    scale = 1.0 / hw
    feat = jnp.zeros_like(b_ref[...]) + b_ref[...]
    for ci in range(c):
        pooled = jnp.sum(x[:, ci * hw:(ci + 1) * hw], axis=1, keepdims=True) * scale
        feat = feat + pooled * w_ref[ci:ci + 1, :]
    ss = jnp.sum(feat * feat, axis=1, keepdims=True)
    xn = feat * (jax.lax.rsqrt(jnp.maximum(ss, 1e-24)) * inv_temperature)
    p = jax.lax.dot_general(xn.astype(jnp.bfloat16), emb_ref[...],
                            (((1,), (1,)), ((), ())),
                            preferred_element_type=jnp.float32)
    m = jnp.max(p, axis=1, keepdims=True)
    lse = jnp.log(jnp.sum(jnp.exp(p - m), axis=1, keepdims=True)) + m
    cols = jax.lax.broadcasted_iota(jnp.int32, p.shape, 1)
    picked = jnp.sum(jnp.where(cols == y_ref[...], p, 0.0), axis=1, keepdims=True)
    ce_ref[...] = lse - picked


def _struc_kernel(emb_slab_ref, emb_full_ref, struc_ref, out_ref):
    gram = jax.lax.dot_general(emb_slab_ref[...], emb_full_ref[...],
                               (((1,), (1,)), ((), ())),
                               preferred_element_type=jnp.float32)
    b = jnp.sqrt(jnp.maximum(2.0 - 2.0 * gram, 0.0))
    s = struc_ref[...]
    out_ref[0, 0, 0] = jnp.sum(s)
    out_ref[0, 0, 1] = jnp.sum(s * s)
    out_ref[0, 0, 2] = jnp.sum(s * b)
    out_ref[0, 0, 3] = jnp.sum(b)
    out_ref[0, 0, 4] = jnp.sum(b * b)


def kernel(x_img, y, w_cnn, b_cnn, emb_raw, w_emb, b_emb, struc):
    N, C, H, W = x_img.shape
    HW = H * W
    K = struc.shape[0]
    Dw = emb_raw.shape[1]
    F = w_cnn.shape[1]
    temperature = 0.1
    struc_weight = 0.5

    tb = _fit_tile(N, 256)
    tk = _fit_tile(K, 512)
    nk = K // tk

    x2d = x_img.reshape(N, C * HW)
    y2d = y.reshape(N, 1).astype(jnp.int32)

    cp_par = pltpu.CompilerParams(dimension_semantics=("parallel",),
                                  vmem_limit_bytes=_VMEM_LIMIT)

    emb = pl.pallas_call(
        _emb_kernel,
        out_shape=jax.ShapeDtypeStruct((K, F), jnp.bfloat16),
        grid=(nk,),
        in_specs=[pl.BlockSpec((tk, Dw), lambda i: (i, 0)),
                  pl.BlockSpec((Dw, F), lambda i: (0, 0)),
                  pl.BlockSpec((1, F), lambda i: (0, 0))],
        out_specs=pl.BlockSpec((tk, F), lambda i: (i, 0)),
        compiler_params=cp_par,
        cost_estimate=pl.CostEstimate(
            flops=2 * K * Dw * F, transcendentals=K,
            bytes_accessed=K * Dw * 4 + Dw * F * 4 + F * 4 + K * F * 2),
    )(emb_raw, w_emb, b_emb.astype(jnp.float32))

    ce_rows = pl.pallas_call(
        functools.partial(_ce_kernel, inv_temperature=1.0 / temperature, c=C, hw=HW),
        out_shape=jax.ShapeDtypeStruct((N, 1), jnp.float32),
        grid=(N // tb,),
        in_specs=[pl.BlockSpec((tb, C * HW), lambda i: (i, 0)),
                  pl.BlockSpec((C, F), lambda i: (0, 0)),
                  pl.BlockSpec((1, F), lambda i: (0, 0)),
                  pl.BlockSpec((K, F), lambda i: (0, 0)),
                  pl.BlockSpec((tb, 1), lambda i: (i, 0))],
        out_specs=pl.BlockSpec((tb, 1), lambda i: (i, 0)),
        compiler_params=cp_par,
        cost_estimate=pl.CostEstimate(
            flops=N * C * HW + 2 * N * F * K, transcendentals=N * K + 2 * N,
            bytes_accessed=N * C * HW * 4 + C * F * 4 + K * F * 2 + N * 8),
    )(x2d, w_cnn.astype(jnp.float32), b_cnn.astype(jnp.float32), emb, y2d)

    parts = pl.pallas_call(
        _struc_kernel,
        out_shape=jax.ShapeDtypeStruct((nk, 1, 5), jnp.float32),
        grid=(nk,),
        in_specs=[pl.BlockSpec((tk, F), lambda i: (i, 0)),
                  pl.BlockSpec((K, F), lambda i: (0, 0)),
                  pl.BlockSpec((tk, K), lambda i: (i, 0))],
        out_specs=pl.BlockSpec((1, 1, 5), lambda i: (i, 0, 0),
                               memory_space=pltpu.MemorySpace.SMEM),
        compiler_params=cp_par,
        cost_estimate=pl.CostEstimate(
            flops=2 * K * K * F + 8 * K * K, transcendentals=K * K,
            bytes_accessed=2 * K * F * 2 + K * K * 4 + 40),
    )(emb, emb, struc)

    sums = jnp.sum(parts, axis=(0, 1))
    s_s, s_ss, s_sb, s_b, s_bb = sums[0], sums[1], sums[2], sums[3], sums[4]
    kk = float(K * K)
    ms = s_s / kk
    mb = s_b / kk
    struc_loss = (s_ss / (ms * ms) - 2.0 * s_sb / (ms * mb) + s_bb / (mb * mb)) / kk
    source_loss = jnp.mean(ce_rows)
    loss = source_loss + struc_weight * struc_loss
    return loss, source_loss, struc_loss


# speedup vs baseline: 1.3852x; 1.1585x over previous
import functools

import jax
import jax.numpy as jnp
from jax.experimental import pallas as pl
from jax.experimental.pallas import tpu as pltpu

_VMEM_LIMIT = 48 * 1024 * 1024


def _fit_tile(dim, pref):
    t = max(1, min(pref, dim))
    while dim % t != 0:
        t //= 2
    return max(t, 1)


def _emb_kernel(raw_ref, w_ref, b_ref, emb_ref):
    raw = raw_ref[...].astype(jnp.bfloat16)
    w = w_ref[...].astype(jnp.bfloat16)
    proj = jnp.dot(raw, w, preferred_element_type=jnp.float32) + b_ref[...]
    ss = jnp.sum(proj * proj, axis=1, keepdims=True)
    emb_ref[...] = (proj * jax.lax.rsqrt(jnp.maximum(ss, 1e-24))).astype(emb_ref.dtype)


def _main_kernel(x_ref, w_ref, b_ref, emb_ref, y_ref, emb_slab_ref, struc_ref,
                 out_ref, *, inv_temperature, c, hw):
    x = x_ref[...]
    scale = 1.0 / hw
    feat = jnp.zeros_like(b_ref[...]) + b_ref[...]
    for ci in range(c):
        pooled = jnp.sum(x[:, ci * hw:(ci + 1) * hw], axis=1, keepdims=True) * scale
        feat = feat + pooled * w_ref[ci:ci + 1, :]
    ss = jnp.sum(feat * feat, axis=1, keepdims=True)
    xn = feat * (jax.lax.rsqrt(jnp.maximum(ss, 1e-24)) * inv_temperature)
    p = jax.lax.dot_general(xn.astype(jnp.bfloat16), emb_ref[...],
                            (((1,), (1,)), ((), ())),
                            preferred_element_type=jnp.float32)
    m = jnp.max(p, axis=1, keepdims=True)
    lse = jnp.log(jnp.sum(jnp.exp(p - m), axis=1, keepdims=True)) + m
    cols = jax.lax.broadcasted_iota(jnp.int32, p.shape, 1)
    picked = jnp.sum(jnp.where(cols == y_ref[...], p, 0.0), axis=1, keepdims=True)
    out_ref[0, 0, 0] = jnp.sum(lse - picked)

    gram = jax.lax.dot_general(emb_slab_ref[...], emb_ref[...],
                               (((1,), (1,)), ((), ())),
                               preferred_element_type=jnp.float32)
    b = jnp.sqrt(jnp.maximum(2.0 - 2.0 * gram, 0.0))
    s = struc_ref[...]
    out_ref[0, 0, 1] = jnp.sum(s)
    out_ref[0, 0, 2] = jnp.sum(s * s)
    out_ref[0, 0, 3] = jnp.sum(s * b)
    out_ref[0, 0, 4] = jnp.sum(b)
    out_ref[0, 0, 5] = jnp.sum(b * b)


def kernel(x_img, y, w_cnn, b_cnn, emb_raw, w_emb, b_emb, struc):
    N, C, H, W = x_img.shape
    HW = H * W
    K = struc.shape[0]
    Dw = emb_raw.shape[1]
    F = w_cnn.shape[1]
    temperature = 0.1
    struc_weight = 0.5

    nb = N // _fit_tile(N, 256)
    tb = N // nb
    tk = _fit_tile(K, -(-K // nb)) if K % nb == 0 else K
    if K // tk > nb:
        tk = K
    nk_steps = K // tk
    ek = _fit_tile(K, 512)

    x2d = x_img.reshape(N, C * HW)
    y2d = y.reshape(N, 1).astype(jnp.int32)

    cp_par = pltpu.CompilerParams(dimension_semantics=("parallel",),
                                  vmem_limit_bytes=_VMEM_LIMIT)

    emb = pl.pallas_call(
        _emb_kernel,
        out_shape=jax.ShapeDtypeStruct((K, F), jnp.bfloat16),
        grid=(K // ek,),
        in_specs=[pl.BlockSpec((ek, Dw), lambda i: (i, 0)),
                  pl.BlockSpec((Dw, F), lambda i: (0, 0)),
                  pl.BlockSpec((1, F), lambda i: (0, 0))],
        out_specs=pl.BlockSpec((ek, F), lambda i: (i, 0)),
        compiler_params=cp_par,
        cost_estimate=pl.CostEstimate(
            flops=2 * K * Dw * F, transcendentals=K,
            bytes_accessed=K * Dw * 4 + Dw * F * 4 + F * 4 + K * F * 2),
    )(emb_raw, w_emb, b_emb.astype(jnp.float32))

    def _slab(i):
        return jnp.minimum(i, nk_steps - 1)

    parts = pl.pallas_call(
        functools.partial(_main_kernel, inv_temperature=1.0 / temperature,
                          c=C, hw=HW),
        out_shape=jax.ShapeDtypeStruct((nb, 1, 8), jnp.float32),
        grid=(nb,),
        in_specs=[pl.BlockSpec((tb, C * HW), lambda i: (i, 0)),
                  pl.BlockSpec((C, F), lambda i: (0, 0)),
                  pl.BlockSpec((1, F), lambda i: (0, 0)),
                  pl.BlockSpec((K, F), lambda i: (0, 0)),
                  pl.BlockSpec((tb, 1), lambda i: (i, 0)),
                  pl.BlockSpec((tk, F), lambda i: (_slab(i), 0)),
                  pl.BlockSpec((tk, K), lambda i: (_slab(i), 0))],
        out_specs=pl.BlockSpec((1, 1, 8), lambda i: (i, 0, 0),
                               memory_space=pltpu.MemorySpace.SMEM),
        compiler_params=cp_par,
        cost_estimate=pl.CostEstimate(
            flops=N * C * HW + 2 * N * F * K + 2 * K * K * F + 8 * K * K,
            transcendentals=N * K + 2 * N + K * K,
            bytes_accessed=(N * C * HW * 4 + C * F * 4 + K * F * 2 + N * 8
                            + K * F * 2 + K * K * 4)),
    )(x2d, w_cnn.astype(jnp.float32), b_cnn.astype(jnp.float32), emb, y2d,
      emb, struc)

    ce_sum = jnp.sum(parts[:, 0, 0])
    dup = jnp.arange(nb) < nk_steps
    sums = jnp.sum(jnp.where(dup[:, None], parts[:, 0, 1:6], 0.0), axis=0)
    s_s, s_ss, s_sb, s_b, s_bb = sums[0], sums[1], sums[2], sums[3], sums[4]
    kk = float(K * K)
    ms = s_s / kk
    mb = s_b / kk
    struc_loss = (s_ss / (ms * ms) - 2.0 * s_sb / (ms * mb) + s_bb / (mb * mb)) / kk
    source_loss = ce_sum / float(N)
    loss = source_loss + struc_weight * struc_loss
    return loss, source_loss, struc_loss
